# Initial kernel scaffold; baseline (speedup 1.0000x reference)
import jax
import jax.numpy as jnp
from jax.experimental import pallas as pl


def kernel(features, edge_index, W0, b0, W1, b1, W2, b2):
    raise NotImplementedError("write your pallas kernel here")


# speedup vs baseline: 11.2613x; 11.2613x over previous
import functools

import jax
import jax.numpy as jnp
from jax import lax
from jax.experimental import pallas as pl
from jax.experimental.pallas import tpu as pltpu
from jax.experimental.pallas import tpu_sc as plsc

N = 100000
E = 1600000
D_IN = 128
D_H = 128
NCLS = 32
ALPHA = 0.1
K_PROP = 10

NC = 2
NS = 16
HALF = NCLS // NC

NP = 102400
PAD_SPREAD = 2048

EP = 1638400
EROWS = EP // 128
TROWS = EROWS // NS
SCH = TROWS // 8

DR = NP // NS
RC = 320
NCH = DR // RC
SUP = 4
NSC = TROWS // SUP

@functools.cache
def _mesh():
    return plsc.VectorSubcoreMesh(core_axis_name="c", subcore_axis_name="s",
                                  num_cores=NC, num_subcores=NS)


def _hist_body(hidx_ref, deg_ref, degsp, idx_v, ones_v, zv):
    c = lax.axis_index("c")
    s = lax.axis_index("s")
    for j in range(8):
        ones_v[pl.ds(j * 16, 16)] = jnp.ones((16,), jnp.float32)

    def _zb(r, _):
        zv[pl.ds(r * 16, 16)] = jnp.zeros((16,), jnp.float32)
        return 0

    lax.fori_loop(0, (NP // NS) // 16, _zb, 0)
    pltpu.sync_copy(zv, degsp.at[pl.ds(s * (NP // NS), NP // NS)])
    plsc.subcore_barrier()

    def _hb(i, _):
        base = s * TROWS + i * 8
        pltpu.sync_copy(hidx_ref.at[c, pl.ds(base, 8)], idx_v)
        for j in range(8):
            pltpu.sync_copy(ones_v, degsp.at[idx_v.at[j]], add=True)
        return 0

    lax.fori_loop(0, SCH, _hb, 0)
    plsc.subcore_barrier()
    pltpu.sync_copy(degsp.at[pl.ds(s * (NP // NS), NP // NS)],
                    deg_ref.at[c, pl.ds(s * (NP // NS), NP // NS)])


@functools.cache
def _hist():
    return pl.kernel(
        _hist_body,
        out_type=jax.ShapeDtypeStruct((NC, NP), jnp.float32),
        mesh=_mesh(),
        compiler_params=pltpu.CompilerParams(use_tc_tiling_on_sc=False),
        scratch_types=[
            pltpu.VMEM_SHARED((NP,), jnp.float32),
            pltpu.VMEM((8, 128), jnp.int32),
            pltpu.VMEM((128,), jnp.float32),
            pltpu.VMEM((NP // NS,), jnp.float32),
        ],
    )


def _prop_body(y_ref, srcg_ref, dst_ref, u_ref, w_ref, out_ref,
               acc, idx_s, idx_d, msg, acc_v, u_v, y_v,
               gsem, ssem):
    c = lax.axis_index("c")
    s = lax.axis_index("s")

    def _zb(r, _):
        y_v[r] = jnp.zeros((HALF,), jnp.float32)
        return 0

    lax.fori_loop(0, RC, _zb, 0)
    for k in range(NCH):
        pltpu.sync_copy(y_v, acc.at[pl.ds(s * DR + k * RC, RC)])
    plsc.subcore_barrier()

    def _eb(i, _):
        base = s * TROWS + i * SUP
        pltpu.sync_copy(srcg_ref.at[c, pl.ds(base, SUP)], idx_s)
        pltpu.sync_copy(dst_ref.at[pl.ds(base, SUP)], idx_d)
        gs = [pltpu.async_copy(y_ref.at[idx_s.at[j]],
                               msg.at[pl.ds(j * 128, 128)], gsem)
              for j in range(SUP)]
        for g in gs:
            g.wait()
        ss = [pltpu.async_copy(msg.at[pl.ds(j * 128, 128)],
                               acc.at[idx_d.at[j]], ssem, add=True)
              for j in range(SUP)]
        for t in ss:
            t.wait()
        return 0

    lax.fori_loop(0, NSC, _eb, 0)
    plsc.subcore_barrier()

    for k in range(NCH):
        r0 = s * DR + k * RC
        pltpu.sync_copy(acc.at[pl.ds(r0, RC)], acc_v)
        pltpu.sync_copy(u_ref.at[pl.ds(r0, RC)], u_v)
        pltpu.sync_copy(w_ref.at[pl.ds(c * NP + r0, RC)],
                        msg.at[pl.ds(0, RC)])

        def _rb(r, _):
            y_v[r] = u_v[r] * acc_v[r] + ALPHA * msg[r]
            return 0

        lax.fori_loop(0, RC, _rb, 0)
        pltpu.sync_copy(y_v, out_ref.at[pl.ds(c * NP + r0, RC)])


@functools.cache
def _prop():
    return pl.kernel(
        _prop_body,
        out_type=jax.ShapeDtypeStruct((NC * NP, HALF), jnp.float32),
        mesh=_mesh(),
        compiler_params=pltpu.CompilerParams(use_tc_tiling_on_sc=False),
        scratch_types=[
            pltpu.VMEM_SHARED((NP, HALF), jnp.float32),
            pltpu.VMEM((SUP, 128), jnp.int32),
            pltpu.VMEM((SUP, 128), jnp.int32),
            pltpu.VMEM((SUP * 128, HALF), jnp.float32),
            pltpu.VMEM((RC, HALF), jnp.float32),
            pltpu.VMEM((RC, HALF), jnp.float32),
            pltpu.VMEM((RC, HALF), jnp.float32),
            pltpu.SemaphoreType.DMA,
            pltpu.SemaphoreType.DMA,
        ],
    )


def _mlp_block(x_ref, w0_ref, b0_ref, w1_ref, b1_ref, w2_ref, b2_ref,
               out_ref):
    h = jnp.dot(x_ref[...], w0_ref[...], preferred_element_type=jnp.float32)
    h = jnp.maximum(h + b0_ref[...], 0.0)
    h = jnp.dot(h, w1_ref[...], preferred_element_type=jnp.float32)
    h = jnp.maximum(h + b1_ref[...], 0.0)
    h = jnp.dot(h, w2_ref[...], preferred_element_type=jnp.float32)
    h = h + b2_ref[...]
    out_ref[0] = h[:, :HALF]
    out_ref[1] = h[:, HALF:]


_MLP_R = 1000


def _mlp(features, W0, b0, W1, b1, W2, b2):
    grid = (N // _MLP_R,)
    return pl.pallas_call(
        _mlp_block,
        grid=grid,
        in_specs=[
            pl.BlockSpec((_MLP_R, D_IN), lambda i: (i, 0)),
            pl.BlockSpec((D_IN, D_H), lambda i: (0, 0)),
            pl.BlockSpec((1, D_H), lambda i: (0, 0)),
            pl.BlockSpec((D_H, D_H), lambda i: (0, 0)),
            pl.BlockSpec((1, D_H), lambda i: (0, 0)),
            pl.BlockSpec((D_H, NCLS), lambda i: (0, 0)),
            pl.BlockSpec((1, NCLS), lambda i: (0, 0)),
        ],
        out_specs=pl.BlockSpec((NC, _MLP_R, HALF), lambda i: (0, i, 0)),
        out_shape=jax.ShapeDtypeStruct((NC, NP, HALF), jnp.float32),
    )(features, W0, b0, W1, b1, W2, b2)


def kernel(features, edge_index, W0, b0, W1, b1, W2, b2):
    src = edge_index[0].astype(jnp.int32)
    dst = edge_index[1].astype(jnp.int32)
    padi = N + (jnp.arange(EP - E, dtype=jnp.int32) % PAD_SPREAD)
    srcp = jnp.concatenate([src, padi])
    dstp = jnp.concatenate([dst, padi])
    hidx = jnp.stack([srcp, dstp]).reshape(NC, EROWS, 128)
    srcg = jnp.stack([srcp, srcp + NP]).reshape(NC, EROWS, 128)
    dst3 = dstp.reshape(EROWS, 128)

    deg = _hist()(hidx)
    d_out = jnp.where(deg[0, :N] > 0, deg[0, :N], 1.0) ** -0.5
    d_in = jnp.where(deg[1, :N] > 0, deg[1, :N], 1.0) ** -0.5
    pad1 = jnp.ones((NP - N,), jnp.float32)
    u_mid = jnp.broadcast_to(
        jnp.concatenate([(1.0 - ALPHA) * d_out * d_in, pad1])[:, None],
        (NP, HALF))
    u_last = jnp.broadcast_to(
        jnp.concatenate([(1.0 - ALPHA) * d_in, pad1])[:, None], (NP, HALF))

    h0f = _mlp(features, W0, b0.reshape(1, -1), W1, b1.reshape(1, -1),
               W2, b2.reshape(1, -1)).reshape(NC * NP, HALF)
    dpad = jnp.concatenate([d_out, jnp.ones((NP - N,), jnp.float32)])
    y0f = h0f * jnp.concatenate([dpad, dpad])[:, None]

    y = y0f
    for t in range(K_PROP):
        last = t == K_PROP - 1
        y = _prop()(y, srcg, dst3, u_last if last else u_mid,
                    h0f if last else y0f)
    return jnp.concatenate([y[:N], y[NP:NP + N]], axis=1)


# speedup vs baseline: 21.3749x; 1.8981x over previous
import functools

import jax
import jax.numpy as jnp
from jax import lax
from jax.experimental import pallas as pl
from jax.experimental.pallas import tpu as pltpu
from jax.experimental.pallas import tpu_sc as plsc

N = 100000
E = 1600000
D_IN = 128
D_H = 128
NCLS = 32
ALPHA = 0.1
K_PROP = 10

NC = 2
NS = 16
HALF = NCLS // NC

NP = 102400
PAD_SPREAD = 2048

EP = 1638400
EROWS = EP // 128
TROWS = EROWS // NS
SCH = TROWS // 8

DR = NP // NS
RC = 256
NCH = DR // RC
SUP = 2
NSC = TROWS // SUP
GRP = NSC // 4

@functools.cache
def _mesh():
    return plsc.VectorSubcoreMesh(core_axis_name="c", subcore_axis_name="s",
                                  num_cores=NC, num_subcores=NS)


def _hist_body(hidx_ref, deg_ref, degsp, idx_v, ones_v, zv):
    c = lax.axis_index("c")
    s = lax.axis_index("s")
    for j in range(8):
        ones_v[pl.ds(j * 16, 16)] = jnp.ones((16,), jnp.float32)

    def _zb(r, _):
        zv[pl.ds(r * 16, 16)] = jnp.zeros((16,), jnp.float32)
        return 0

    lax.fori_loop(0, (NP // NS) // 16, _zb, 0)
    pltpu.sync_copy(zv, degsp.at[pl.ds(s * (NP // NS), NP // NS)])
    plsc.subcore_barrier()

    def _hb(i, _):
        base = s * TROWS + i * 8
        pltpu.sync_copy(hidx_ref.at[c, pl.ds(base, 8)], idx_v)
        for j in range(8):
            pltpu.sync_copy(ones_v, degsp.at[idx_v.at[j]], add=True)
        return 0

    lax.fori_loop(0, SCH, _hb, 0)
    plsc.subcore_barrier()
    pltpu.sync_copy(degsp.at[pl.ds(s * (NP // NS), NP // NS)],
                    deg_ref.at[c, pl.ds(s * (NP // NS), NP // NS)])


@functools.cache
def _hist():
    return pl.kernel(
        _hist_body,
        out_type=jax.ShapeDtypeStruct((NC, NP), jnp.float32),
        mesh=_mesh(),
        compiler_params=pltpu.CompilerParams(use_tc_tiling_on_sc=False),
        scratch_types=[
            pltpu.VMEM_SHARED((NP,), jnp.float32),
            pltpu.VMEM((8, 128), jnp.int32),
            pltpu.VMEM((128,), jnp.float32),
            pltpu.VMEM((NP // NS,), jnp.float32),
        ],
    )


def _prop_body(y_ref, srcg_ref, dst_ref, u_ref, w_ref, out_ref,
               acc, is0, is1, is2, is3, id0, id1, id2, id3,
               m0, m1, m2, m3, acc_v, u_v,
               gs0, gs1, gs2, gs3, ss0, ss1, ss2, ss3,
               im0, im1, im2, im3):
    c = lax.axis_index("c")
    s = lax.axis_index("s")
    IS = [is0, is1, is2, is3]
    ID = [id0, id1, id2, id3]
    MS = [m0, m1, m2, m3]
    GS = [gs0, gs1, gs2, gs3]
    SS = [ss0, ss1, ss2, ss3]
    IM = [im0, im1, im2, im3]
    base = s * TROWS

    def stage(n, q, sync=False):
        ssl = srcg_ref.at[c, pl.ds(base + n * SUP, SUP)]
        dsl = dst_ref.at[pl.ds(base + n * SUP, SUP)]
        if sync:
            pltpu.sync_copy(ssl, IS[q])
            pltpu.sync_copy(dsl, ID[q])
        else:
            pltpu.async_copy(ssl, IS[q], IM[q])
            pltpu.async_copy(dsl, ID[q], IM[q])

    def wait_idx(q):
        pltpu.make_async_copy(srcg_ref.at[c, pl.ds(0, SUP)], IS[q],
                              IM[q]).wait()
        pltpu.make_async_copy(dst_ref.at[pl.ds(0, SUP)], ID[q],
                              IM[q]).wait()

    def gath(q):
        for j in range(SUP):
            pltpu.async_copy(y_ref.at[IS[q].at[j]],
                             MS[q].at[pl.ds(j * 128, 128)], GS[q])

    def wait_gath(q):
        for j in range(SUP):
            pltpu.make_async_copy(y_ref.at[IS[q].at[j]],
                                  MS[q].at[pl.ds(j * 128, 128)],
                                  GS[q]).wait()

    def scat(q):
        for j in range(SUP):
            pltpu.async_copy(MS[q].at[pl.ds(j * 128, 128)],
                             acc.at[ID[q].at[j]], SS[q], add=True)

    def wait_scat(q):
        for j in range(SUP):
            pltpu.make_async_copy(MS[q].at[pl.ds(j * 128, 128)],
                                  acc.at[ID[q].at[j]], SS[q]).wait()

    def _zb(r, _):
        m2[r] = jnp.zeros((HALF,), jnp.float32)
        return 0

    lax.fori_loop(0, RC, _zb, 0)
    zcs = [pltpu.async_copy(m2, acc.at[pl.ds(s * DR + k * RC, RC)], ss0)
           for k in range(NCH)]
    for z in zcs:
        z.wait()
    plsc.subcore_barrier()

    stage(0, 0, sync=True)
    stage(1, 1, sync=True)
    gath(0)
    gath(1)
    stage(2, 2, sync=True)
    stage(3, 3, sync=True)

    def _grp(g, _):
        for q in range(4):
            m = 4 * g + q
            q2 = (q + 2) % 4
            wait_gath(q)
            scat(q)

            @pl.when(g < GRP - 1)
            def _():
                stage(m + 4, q)

            if q < 2:
                @pl.when(g > 0)
                def _():
                    wait_scat(q2)
                    wait_idx(q2)

                gath(q2)
            else:
                @pl.when(g < GRP - 1)
                def _():
                    wait_scat(q2)
                    wait_idx(q2)
                    gath(q2)
        return 0

    lax.fori_loop(0, GRP, _grp, 0)
    for q in range(4):
        wait_scat(q)
    plsc.subcore_barrier()

    def _dr(k, _):
        r0 = s * DR + k * RC
        pltpu.sync_copy(acc.at[pl.ds(r0, RC)], acc_v)
        pltpu.sync_copy(u_ref.at[pl.ds(r0, RC)], u_v)
        pltpu.sync_copy(w_ref.at[pl.ds(c * NP + r0, RC)], m0)

        def _rb(r, _):
            m1[r] = u_v[r] * acc_v[r] + ALPHA * m0[r]
            return 0

        lax.fori_loop(0, RC, _rb, 0)
        pltpu.sync_copy(m1, out_ref.at[pl.ds(c * NP + r0, RC)])
        return 0

    lax.fori_loop(0, NCH, _dr, 0)


@functools.cache
def _prop():
    return pl.kernel(
        _prop_body,
        out_type=jax.ShapeDtypeStruct((NC * NP, HALF), jnp.float32),
        mesh=_mesh(),
        compiler_params=pltpu.CompilerParams(use_tc_tiling_on_sc=False),
        scratch_types=(
            [pltpu.VMEM_SHARED((NP, HALF), jnp.float32)]
            + [pltpu.VMEM((SUP, 128), jnp.int32) for _ in range(8)]
            + [pltpu.VMEM((SUP * 128, HALF), jnp.float32) for _ in range(4)]
            + [pltpu.VMEM((RC, HALF), jnp.float32) for _ in range(2)]
            + [pltpu.SemaphoreType.DMA for _ in range(12)]
        ),
    )


def _mlp_block(x_ref, w0_ref, b0_ref, w1_ref, b1_ref, w2_ref, b2_ref,
               out_ref):
    h = jnp.dot(x_ref[...], w0_ref[...], preferred_element_type=jnp.float32)
    h = jnp.maximum(h + b0_ref[...], 0.0)
    h = jnp.dot(h, w1_ref[...], preferred_element_type=jnp.float32)
    h = jnp.maximum(h + b1_ref[...], 0.0)
    h = jnp.dot(h, w2_ref[...], preferred_element_type=jnp.float32)
    h = h + b2_ref[...]
    out_ref[0] = h[:, :HALF]
    out_ref[1] = h[:, HALF:]


_MLP_R = 1000


def _mlp(features, W0, b0, W1, b1, W2, b2):
    grid = (N // _MLP_R,)
    return pl.pallas_call(
        _mlp_block,
        grid=grid,
        in_specs=[
            pl.BlockSpec((_MLP_R, D_IN), lambda i: (i, 0)),
            pl.BlockSpec((D_IN, D_H), lambda i: (0, 0)),
            pl.BlockSpec((1, D_H), lambda i: (0, 0)),
            pl.BlockSpec((D_H, D_H), lambda i: (0, 0)),
            pl.BlockSpec((1, D_H), lambda i: (0, 0)),
            pl.BlockSpec((D_H, NCLS), lambda i: (0, 0)),
            pl.BlockSpec((1, NCLS), lambda i: (0, 0)),
        ],
        out_specs=pl.BlockSpec((NC, _MLP_R, HALF), lambda i: (0, i, 0)),
        out_shape=jax.ShapeDtypeStruct((NC, NP, HALF), jnp.float32),
    )(features, W0, b0, W1, b1, W2, b2)


def kernel(features, edge_index, W0, b0, W1, b1, W2, b2):
    src = edge_index[0].astype(jnp.int32)
    dst = edge_index[1].astype(jnp.int32)
    padi = N + (jnp.arange(EP - E, dtype=jnp.int32) % PAD_SPREAD)
    srcp = jnp.concatenate([src, padi])
    dstp = jnp.concatenate([dst, padi])
    hidx = jnp.stack([srcp, dstp]).reshape(NC, EROWS, 128)
    srcg = jnp.stack([srcp, srcp + NP]).reshape(NC, EROWS, 128)
    dst3 = dstp.reshape(EROWS, 128)

    deg = _hist()(hidx)
    d_out = jnp.where(deg[0, :N] > 0, deg[0, :N], 1.0) ** -0.5
    d_in = jnp.where(deg[1, :N] > 0, deg[1, :N], 1.0) ** -0.5
    pad1 = jnp.ones((NP - N,), jnp.float32)
    u_mid = jnp.broadcast_to(
        jnp.concatenate([(1.0 - ALPHA) * d_out * d_in, pad1])[:, None],
        (NP, HALF))
    u_last = jnp.broadcast_to(
        jnp.concatenate([(1.0 - ALPHA) * d_in, pad1])[:, None], (NP, HALF))

    h0f = _mlp(features, W0, b0.reshape(1, -1), W1, b1.reshape(1, -1),
               W2, b2.reshape(1, -1)).reshape(NC * NP, HALF)
    dpad = jnp.concatenate([d_out, jnp.ones((NP - N,), jnp.float32)])
    y0f = h0f * jnp.concatenate([dpad, dpad])[:, None]

    y = y0f
    for t in range(K_PROP):
        last = t == K_PROP - 1
        y = _prop()(y, srcg, dst3, u_last if last else u_mid,
                    h0f if last else y0f)
    return jnp.concatenate([y[:N], y[NP:NP + N]], axis=1)


# speedup vs baseline: 23.3095x; 1.0905x over previous
import functools

import jax
import jax.numpy as jnp
from jax import lax
from jax.experimental import pallas as pl
from jax.experimental.pallas import tpu as pltpu
from jax.experimental.pallas import tpu_sc as plsc

N = 100000
E = 1600000
D_IN = 128
D_H = 128
NCLS = 32
ALPHA = 0.1
K_PROP = 10

NC = 2
NS = 16
HALF = NCLS // NC

NP = 102400
PAD_SPREAD = 2048

EP = 1638400
EROWS = EP // 128
TROWS = EROWS // NS
SCH = TROWS // 4

DR = NP // NS
RC = 256
NCH = DR // RC
SUP = 2
NSC = TROWS // SUP
GRP = NSC // 4

@functools.cache
def _mesh():
    return plsc.VectorSubcoreMesh(core_axis_name="c", subcore_axis_name="s",
                                  num_cores=NC, num_subcores=NS)


def _hist_body(hidx_ref, deg_ref, degsp, idx_v, ones_v, zv, hsem, ssem):
    c = lax.axis_index("c")
    s = lax.axis_index("s")
    for j in range(8):
        ones_v[pl.ds(j * 16, 16)] = jnp.ones((16,), jnp.float32)

    def _zb(r, _):
        zv[pl.ds(r * 16, 16)] = jnp.zeros((16,), jnp.float32)
        return 0

    lax.fori_loop(0, (NP // NS) // 16, _zb, 0)
    pltpu.sync_copy(zv, degsp.at[pl.ds(s * (NP // NS), NP // NS)])
    plsc.subcore_barrier()

    base = s * TROWS
    GH = SCH // 4

    def _stg(n, q, sync=False):
        sl = hidx_ref.at[c, pl.ds(base + n * 4, 4)]
        if sync:
            pltpu.sync_copy(sl, idx_v.at[pl.ds(q * 4, 4)])
        else:
            pltpu.async_copy(sl, idx_v.at[pl.ds(q * 4, 4)], hsem.at[q])

    def _wstg(q):
        pltpu.make_async_copy(hidx_ref.at[c, pl.ds(0, 4)],
                              idx_v.at[pl.ds(q * 4, 4)], hsem.at[q]).wait()

    def _sc4(q):
        for j in range(4):
            pltpu.async_copy(ones_v, degsp.at[idx_v.at[q * 4 + j]],
                             ssem.at[q], add=True)

    def _wsc4(q):
        for j in range(4):
            pltpu.make_async_copy(ones_v, degsp.at[idx_v.at[q * 4 + j]],
                                  ssem.at[q]).wait()

    for q in range(4):
        _stg(q, q, sync=True)

    def _hb(g, _):
        for q in range(4):
            m = 4 * g + q
            q2 = (q + 2) % 4

            @pl.when(g > 0)
            def _():
                _wstg(q)

            _sc4(q)
            if q < 2:
                @pl.when(g > 0)
                def _():
                    _wsc4(q2)
                    _stg(m + 2, q2)
            else:
                @pl.when(g < GH - 1)
                def _():
                    _wsc4(q2)
                    _stg(m + 2, q2)
        return 0

    lax.fori_loop(0, GH, _hb, 0)
    for q in range(4):
        _wsc4(q)
    plsc.subcore_barrier()
    pltpu.sync_copy(degsp.at[pl.ds(s * (NP // NS), NP // NS)],
                    deg_ref.at[c, pl.ds(s * (NP // NS), NP // NS)])


@functools.cache
def _hist():
    return pl.kernel(
        _hist_body,
        out_type=jax.ShapeDtypeStruct((NC, NP), jnp.float32),
        mesh=_mesh(),
        compiler_params=pltpu.CompilerParams(use_tc_tiling_on_sc=False),
        scratch_types=[
            pltpu.VMEM_SHARED((NP,), jnp.float32),
            pltpu.VMEM((16, 128), jnp.int32),
            pltpu.VMEM((128,), jnp.float32),
            pltpu.VMEM((NP // NS,), jnp.float32),
            pltpu.SemaphoreType.DMA((4,)),
            pltpu.SemaphoreType.DMA((4,)),
        ],
    )


def _prop_body(y_ref, srcg_ref, dst_ref, u_ref, wi_ref, out_ref,
               acc, is0, is1, is2, is3, id0, id1, id2, id3,
               m0, m1, m2, m3, acc_v, u_v,
               gs0, gs1, gs2, gs3, ss0, ss1, ss2, ss3,
               im0, im1, im2, im3):
    c = lax.axis_index("c")
    s = lax.axis_index("s")
    IS = [is0, is1, is2, is3]
    ID = [id0, id1, id2, id3]
    MS = [m0, m1, m2, m3]
    GS = [gs0, gs1, gs2, gs3]
    SS = [ss0, ss1, ss2, ss3]
    IM = [im0, im1, im2, im3]
    base = s * TROWS

    def stage(n, q, sync=False):
        ssl = srcg_ref.at[c, pl.ds(base + n * SUP, SUP)]
        dsl = dst_ref.at[pl.ds(base + n * SUP, SUP)]
        if sync:
            pltpu.sync_copy(ssl, IS[q])
            pltpu.sync_copy(dsl, ID[q])
        else:
            pltpu.async_copy(ssl, IS[q], IM[q])
            pltpu.async_copy(dsl, ID[q], IM[q])

    def wait_idx(q):
        pltpu.make_async_copy(srcg_ref.at[c, pl.ds(0, SUP)], IS[q],
                              IM[q]).wait()
        pltpu.make_async_copy(dst_ref.at[pl.ds(0, SUP)], ID[q],
                              IM[q]).wait()

    def gath(q):
        for j in range(SUP):
            pltpu.async_copy(y_ref.at[IS[q].at[j]],
                             MS[q].at[pl.ds(j * 128, 128)], GS[q])

    def wait_gath(q):
        for j in range(SUP):
            pltpu.make_async_copy(y_ref.at[IS[q].at[j]],
                                  MS[q].at[pl.ds(j * 128, 128)],
                                  GS[q]).wait()

    def scat(q):
        for j in range(SUP):
            pltpu.async_copy(MS[q].at[pl.ds(j * 128, 128)],
                             acc.at[ID[q].at[j]], SS[q], add=True)

    def wait_scat(q):
        for j in range(SUP):
            pltpu.make_async_copy(MS[q].at[pl.ds(j * 128, 128)],
                                  acc.at[ID[q].at[j]], SS[q]).wait()

    zcs = [pltpu.async_copy(wi_ref.at[pl.ds(c * NP + s * DR + k * RC, RC)],
                            acc.at[pl.ds(s * DR + k * RC, RC)], ss0)
           for k in range(NCH)]
    for z in zcs:
        z.wait()
    plsc.subcore_barrier()

    stage(0, 0, sync=True)
    stage(1, 1, sync=True)
    gath(0)
    gath(1)
    stage(2, 2, sync=True)
    stage(3, 3, sync=True)

    def _grp(g, _):
        for q in range(4):
            m = 4 * g + q
            q2 = (q + 2) % 4
            wait_gath(q)
            scat(q)

            @pl.when(g < GRP - 1)
            def _():
                stage(m + 4, q)

            if q < 2:
                @pl.when(g > 0)
                def _():
                    wait_scat(q2)
                    wait_idx(q2)

                gath(q2)
            else:
                @pl.when(g < GRP - 1)
                def _():
                    wait_scat(q2)
                    wait_idx(q2)
                    gath(q2)
        return 0

    lax.fori_loop(0, GRP, _grp, 0)
    for q in range(4):
        wait_scat(q)
    plsc.subcore_barrier()

    AB = [m0, m1]
    UB = [m2, m3]
    YB = [acc_v, u_v]
    GSd = [gs0, gs1]
    IMd = [im0, im1]
    SSd = [ss0, ss1]

    def _pref(n, p):
        r0 = s * DR + n * RC
        pltpu.async_copy(acc.at[pl.ds(r0, RC)], AB[p], GSd[p])
        pltpu.async_copy(u_ref.at[pl.ds(r0, RC)], UB[p], IMd[p])

    def _wpref(p):
        pltpu.make_async_copy(acc.at[pl.ds(0, RC)], AB[p], GSd[p]).wait()
        pltpu.make_async_copy(u_ref.at[pl.ds(0, RC)], UB[p], IMd[p]).wait()

    def _ywait(p):
        pltpu.make_async_copy(YB[p], out_ref.at[pl.ds(0, RC)],
                              SSd[p]).wait()

    def _compute(p):
        def _rb(r, _):
            YB[p][r] = UB[p][r] * AB[p][r]
            return 0

        lax.fori_loop(0, RC, _rb, 0)

    def _ywrite(n, p):
        pltpu.async_copy(YB[p], out_ref.at[pl.ds(c * NP + s * DR + n * RC,
                                                 RC)], SSd[p])

    _pref(0, 0)

    def _dg(g, _):
        for p in range(2):
            n = 2 * g + p
            _wpref(p)
            _pref(n + 1, 1 - p)

            @pl.when(g > 0)
            def _():
                _ywait(p)

            _compute(p)
            _ywrite(n, p)
        return 0

    lax.fori_loop(0, (NCH - 1) // 2, _dg, 0)
    _wpref(0)
    _ywait(0)
    _compute(0)
    _ywrite(NCH - 1, 0)
    _ywait(1)
    _ywait(0)


@functools.cache
def _prop():
    return pl.kernel(
        _prop_body,
        out_type=jax.ShapeDtypeStruct((NC * NP, HALF), jnp.float32),
        mesh=_mesh(),
        compiler_params=pltpu.CompilerParams(use_tc_tiling_on_sc=False),
        scratch_types=(
            [pltpu.VMEM_SHARED((NP, HALF), jnp.float32)]
            + [pltpu.VMEM((SUP, 128), jnp.int32) for _ in range(8)]
            + [pltpu.VMEM((SUP * 128, HALF), jnp.float32) for _ in range(4)]
            + [pltpu.VMEM((RC, HALF), jnp.float32) for _ in range(2)]
            + [pltpu.SemaphoreType.DMA for _ in range(12)]
        ),
    )


def _mlp_block(x_ref, w0_ref, b0_ref, w1_ref, b1_ref, w2_ref, b2_ref,
               out_ref):
    h = jnp.dot(x_ref[...], w0_ref[...], preferred_element_type=jnp.float32)
    h = jnp.maximum(h + b0_ref[...], 0.0)
    h = jnp.dot(h, w1_ref[...], preferred_element_type=jnp.float32)
    h = jnp.maximum(h + b1_ref[...], 0.0)
    h = jnp.dot(h, w2_ref[...], preferred_element_type=jnp.float32)
    h = h + b2_ref[...]
    out_ref[0] = h[:, :HALF]
    out_ref[1] = h[:, HALF:]


_MLP_R = 1000


def _mlp(features, W0, b0, W1, b1, W2, b2):
    grid = (N // _MLP_R,)
    return pl.pallas_call(
        _mlp_block,
        grid=grid,
        in_specs=[
            pl.BlockSpec((_MLP_R, D_IN), lambda i: (i, 0)),
            pl.BlockSpec((D_IN, D_H), lambda i: (0, 0)),
            pl.BlockSpec((1, D_H), lambda i: (0, 0)),
            pl.BlockSpec((D_H, D_H), lambda i: (0, 0)),
            pl.BlockSpec((1, D_H), lambda i: (0, 0)),
            pl.BlockSpec((D_H, NCLS), lambda i: (0, 0)),
            pl.BlockSpec((1, NCLS), lambda i: (0, 0)),
        ],
        out_specs=pl.BlockSpec((NC, _MLP_R, HALF), lambda i: (0, i, 0)),
        out_shape=jax.ShapeDtypeStruct((NC, NP, HALF), jnp.float32),
    )(features, W0, b0, W1, b1, W2, b2)


def kernel(features, edge_index, W0, b0, W1, b1, W2, b2):
    src = edge_index[0].astype(jnp.int32)
    dst = edge_index[1].astype(jnp.int32)
    padi = N + (jnp.arange(EP - E, dtype=jnp.int32) % PAD_SPREAD)
    srcp = jnp.concatenate([src, padi])
    dstp = jnp.concatenate([dst, padi])
    hidx = jnp.stack([srcp, dstp]).reshape(NC, EROWS, 128)
    srcg = jnp.stack([srcp, srcp + NP]).reshape(NC, EROWS, 128)
    dst3 = dstp.reshape(EROWS, 128)

    deg = _hist()(hidx)
    d_out = jnp.where(deg[0, :N] > 0, deg[0, :N], 1.0) ** -0.5
    d_in = jnp.where(deg[1, :N] > 0, deg[1, :N], 1.0) ** -0.5
    pad1 = jnp.ones((NP - N,), jnp.float32)
    un_mid = jnp.concatenate([(1.0 - ALPHA) * d_out * d_in, pad1])
    un_last = jnp.concatenate([(1.0 - ALPHA) * d_in, pad1])
    u_mid = jnp.broadcast_to(un_mid[:, None], (NP, HALF))
    u_last = jnp.broadcast_to(un_last[:, None], (NP, HALF))

    h0f = _mlp(features, W0, b0.reshape(1, -1), W1, b1.reshape(1, -1),
               W2, b2.reshape(1, -1)).reshape(NC * NP, HALF)
    dpad = jnp.concatenate([d_out, pad1])
    y0f = h0f * jnp.concatenate([dpad, dpad])[:, None]
    wi_mid = (ALPHA * y0f) / jnp.concatenate([un_mid, un_mid])[:, None]
    wi_last = (ALPHA * h0f) / jnp.concatenate([un_last, un_last])[:, None]

    y = y0f
    for t in range(K_PROP):
        last = t == K_PROP - 1
        y = _prop()(y, srcg, dst3, u_last if last else u_mid,
                    wi_last if last else wi_mid)
    return jnp.concatenate([y[:N], y[NP:NP + N]], axis=1)
